# baseline (speedup 1.0000x reference)
import functools

import jax
import jax.numpy as jnp
from jax import lax
from jax.experimental import pallas as pl
from jax.experimental.pallas import tpu as pltpu
from jax.experimental.pallas import tpu_sc as plsc

N = 10000
D = 128
O = 128
NNZ = 320000
L = 16
NC = 2
NS = 16
NW = NC * NS
CPT = O // NW
CH = 6400
NCHUNK = NNZ // CH
NGRP = CH // L
UNROLL = 4
NBLK = 500


def _stream_chunks(rows_hbm, cols_hbm, vals_hbm, bufs, process):
  def start(k, b):
    rbuf, cbuf, vbuf, sem = bufs[b]
    sl = pl.ds(k * CH, CH)
    pltpu.async_copy(rows_hbm.at[sl], rbuf, sem)
    pltpu.async_copy(cols_hbm.at[sl], cbuf, sem)
    pltpu.async_copy(vals_hbm.at[sl], vbuf, sem)

  def drain(b):
    rbuf, cbuf, vbuf, sem = bufs[b]
    pltpu.make_async_copy(rows_hbm.at[pl.ds(0, CH)], rbuf, sem).wait()
    pltpu.make_async_copy(cols_hbm.at[pl.ds(0, CH)], cbuf, sem).wait()
    pltpu.make_async_copy(vals_hbm.at[pl.ds(0, CH)], vbuf, sem).wait()

  start(0, 0)
  start(1, 1)

  def step(k2, carry):
    k = k2 * 2
    drain(0)
    process(0)

    @pl.when(k + 2 < NCHUNK)
    def _():
      start(k + 2, 0)

    drain(1)
    process(1)

    @pl.when(k + 3 < NCHUNK)
    def _():
      start(k + 3, 1)
    return carry
  lax.fori_loop(0, NCHUNK // 2, step, 0)


def _densify_body(fr, fc, fv, fb_hbm, acc, rb0, cb0, vb0, rb1, cb1, vb1,
                  sem0, sem1):
  wid = lax.axis_index("s") * NC + lax.axis_index("c")
  colbase = (wid * CPT).astype(jnp.int32)
  bufs = ((rb0, cb0, vb0, sem0), (rb1, cb1, vb1, sem1))

  @plsc.parallel_loop(0, N * CPT // L, unroll=UNROLL)
  def _zero(i):
    acc[pl.ds(i * L, L)] = jnp.zeros((L,), jnp.float32)

  def process(b):
    rbuf, cbuf, vbuf, _ = bufs[b]

    @plsc.parallel_loop(0, NGRP, unroll=UNROLL)
    def _grp(g):
      sl = pl.ds(g * L, L)
      r = rbuf[sl]
      c = cbuf[sl] - colbase
      v = vbuf[sl]
      mask = (c >= 0) & (c < CPT)
      plsc.addupdate_scatter(acc, [c * N + r], v, mask=mask)

  _stream_chunks(fr, fc, fv, bufs, process)
  pltpu.sync_copy(acc, fb_hbm.at[pl.ds(colbase * N, CPT * N)])


def _matmul_kernel(w_ref, fb_ref, o_ref):
  o_ref[...] = lax.dot_general(
      w_ref[...], fb_ref[...], (((0,), (0,)), ((), ())),
      precision=lax.Precision.HIGHEST,
      preferred_element_type=jnp.float32)


def _spmm_body(ar, ac, av, xwt_hbm, out_hbm, xw, ob,
               rb0, cb0, vb0, rb1, cb1, vb1, sem0, sem1, xsem):
  wid = lax.axis_index("s") * NC + lax.axis_index("c")
  colbase = wid * CPT
  bufs = ((rb0, cb0, vb0, sem0), (rb1, cb1, vb1, sem1))

  xcp = pltpu.async_copy(xwt_hbm.at[pl.ds(colbase * N, CPT * N)], xw, xsem)

  @plsc.parallel_loop(0, N * CPT // L, unroll=UNROLL)
  def _zero(i):
    ob[pl.ds(i * L, L)] = jnp.zeros((L,), jnp.float32)

  xcp.wait()

  goffs = [j * N for j in range(CPT)]

  def process(b):
    rbuf, cbuf, vbuf, _ = bufs[b]

    @plsc.parallel_loop(0, NGRP, unroll=UNROLL)
    def _grp(g):
      sl = pl.ds(g * L, L)
      r = rbuf[sl]
      c = cbuf[sl]
      v = vbuf[sl]
      prods = [v * plsc.load_gather(xw, [c + goffs[j]]) for j in range(CPT)]
      for j in range(CPT):
        plsc.addupdate_scatter(ob, [r + goffs[j]], prods[j])

  _stream_chunks(ar, ac, av, bufs, process)

  @plsc.parallel_loop(0, N * CPT // L, unroll=UNROLL)
  def _relu(i):
    sl = pl.ds(i * L, L)
    ob[sl] = jnp.maximum(ob[sl], 0.0)

  pltpu.sync_copy(ob, out_hbm.at[wid])


@functools.partial(jax.jit)
def _run(fr, fc, fv, ar, ac, av, weight):
  mesh = plsc.VectorSubcoreMesh(core_axis_name="c", subcore_axis_name="s")
  chunk_scratch = [
      pltpu.VMEM((CH,), jnp.int32),
      pltpu.VMEM((CH,), jnp.int32),
      pltpu.VMEM((CH,), jnp.float32),
      pltpu.VMEM((CH,), jnp.int32),
      pltpu.VMEM((CH,), jnp.int32),
      pltpu.VMEM((CH,), jnp.float32),
      pltpu.SemaphoreType.DMA,
      pltpu.SemaphoreType.DMA,
  ]

  densify = pl.kernel(
      _densify_body,
      out_type=jax.ShapeDtypeStruct((D * N,), jnp.float32),
      mesh=mesh,
      scratch_types=[pltpu.VMEM((N * CPT,), jnp.float32)] + chunk_scratch,
      compiler_params=pltpu.CompilerParams(needs_layout_passes=False),
  )
  fb = densify(fr, fc, fv)

  xwt = pl.pallas_call(
      _matmul_kernel,
      out_shape=jax.ShapeDtypeStruct((O, N), jnp.float32),
  )(weight, fb.reshape(D, N))

  spmm = pl.kernel(
      _spmm_body,
      out_type=jax.ShapeDtypeStruct((NW, N * CPT), jnp.float32),
      mesh=mesh,
      scratch_types=[
          pltpu.VMEM((N * CPT,), jnp.float32),
          pltpu.VMEM((N * CPT,), jnp.float32),
      ] + chunk_scratch + [pltpu.SemaphoreType.DMA],
      compiler_params=pltpu.CompilerParams(needs_layout_passes=False),
  )
  blocks = spmm(ar, ac, av, xwt.reshape(-1))
  return blocks.reshape(NW, CPT, N).transpose(2, 0, 1).reshape(N, O)


def kernel(feat_rows, feat_cols, feat_values, adj_row, adj_col, adj_values,
           weight):
  return _run(feat_rows, feat_cols, feat_values,
              adj_row, adj_col, adj_values, weight)


# speedup vs baseline: 1.0847x; 1.0847x over previous
import functools

import jax
import jax.numpy as jnp
from jax import lax
from jax.experimental import pallas as pl
from jax.experimental.pallas import tpu as pltpu
from jax.experimental.pallas import tpu_sc as plsc

N = 10000
D = 128
O = 128
NNZ = 320000
L = 16
NC = 2
NS = 16
NW = NC * NS
CPT = O // NW
DPT = D // NS
UNROLL = 4

CH = 6400
NCHUNK = NNZ // CH
ECH = 8000
ENCHUNK = (NNZ // NC) // ECH


def _stream_chunks(rows_hbm, cols_hbm, vals_hbm, base, ch, nchunk, bufs,
                   process):
  def start(k, b):
    rbuf, cbuf, vbuf, sem = bufs[b]
    sl = pl.ds(base + k * ch, ch)
    pltpu.async_copy(rows_hbm.at[sl], rbuf, sem)
    pltpu.async_copy(cols_hbm.at[sl], cbuf, sem)
    pltpu.async_copy(vals_hbm.at[sl], vbuf, sem)

  def drain(b):
    rbuf, cbuf, vbuf, sem = bufs[b]
    pltpu.make_async_copy(rows_hbm.at[pl.ds(0, ch)], rbuf, sem).wait()
    pltpu.make_async_copy(cols_hbm.at[pl.ds(0, ch)], cbuf, sem).wait()
    pltpu.make_async_copy(vals_hbm.at[pl.ds(0, ch)], vbuf, sem).wait()

  start(0, 0)
  start(1, 1)

  def step(k2, carry):
    k = k2 * 2
    drain(0)
    process(0)

    @pl.when(k + 2 < nchunk)
    def _():
      start(k + 2, 0)

    drain(1)
    process(1)

    @pl.when(k + 3 < nchunk)
    def _():
      start(k + 3, 1)
    return carry
  lax.fori_loop(0, nchunk // 2, step, 0)


def _densify_body(fr, fc, fv, fb_hbm, acc, rb0, cb0, vb0, rb1, cb1, vb1,
                  sem0, sem1):
  cid = lax.axis_index("c")
  sid = lax.axis_index("s")
  colbase = (sid * DPT).astype(jnp.int32)
  ebase = cid * (NNZ // NC)
  bufs = ((rb0, cb0, vb0, sem0), (rb1, cb1, vb1, sem1))

  @plsc.parallel_loop(0, N * DPT // L, unroll=UNROLL)
  def _zero(i):
    acc[pl.ds(i * L, L)] = jnp.zeros((L,), jnp.float32)

  def process(b):
    rbuf, cbuf, vbuf, _ = bufs[b]

    @plsc.parallel_loop(0, ECH // L, unroll=UNROLL)
    def _grp(g):
      sl = pl.ds(g * L, L)
      r = rbuf[sl]
      c2 = cbuf[sl] - colbase
      v = vbuf[sl]
      own = (c2 >= 0) & (c2 < DPT)
      plsc.addupdate_scatter(acc, [c2 * N + r], v, mask=own)

  _stream_chunks(fr, fc, fv, ebase, ECH, ENCHUNK, bufs, process)
  pltpu.sync_copy(acc, fb_hbm.at[pl.ds(cid * D * N + colbase * N, DPT * N)])


def _matmul_kernel(w_ref, fb_ref, o_ref):
  fb = fb_ref[0] + fb_ref[1]
  o_ref[...] = lax.dot_general(
      w_ref[...], fb, (((0,), (0,)), ((), ())),
      precision=lax.Precision.HIGHEST,
      preferred_element_type=jnp.float32)


def _spmm_body(ar, ac, av, xwt_hbm, out_hbm, xw, ob,
               rb0, cb0, vb0, rb1, cb1, vb1, sem0, sem1, xsem):
  wid = lax.axis_index("s") * NC + lax.axis_index("c")
  colbase = wid * CPT
  bufs = ((rb0, cb0, vb0, sem0), (rb1, cb1, vb1, sem1))

  xcp = pltpu.async_copy(xwt_hbm.at[pl.ds(colbase * N, CPT * N)], xw, xsem)

  @plsc.parallel_loop(0, N * CPT // L, unroll=UNROLL)
  def _zero(i):
    ob[pl.ds(i * L, L)] = jnp.zeros((L,), jnp.float32)

  xcp.wait()

  goffs = [j * N for j in range(CPT)]

  def process(b):
    rbuf, cbuf, vbuf, _ = bufs[b]

    @plsc.parallel_loop(0, CH // L, unroll=UNROLL)
    def _grp(g):
      sl = pl.ds(g * L, L)
      r = rbuf[sl]
      c = cbuf[sl]
      v = vbuf[sl]
      prods = [v * plsc.load_gather(xw, [c + goffs[j]]) for j in range(CPT)]
      for j in range(CPT):
        plsc.addupdate_scatter(ob, [r + goffs[j]], prods[j])

  _stream_chunks(ar, ac, av, 0, CH, NCHUNK, bufs, process)

  @plsc.parallel_loop(0, N * CPT // L, unroll=UNROLL)
  def _relu(i):
    sl = pl.ds(i * L, L)
    ob[sl] = jnp.maximum(ob[sl], 0.0)

  pltpu.sync_copy(ob, out_hbm.at[wid])


@functools.partial(jax.jit)
def _run(fr, fc, fv, ar, ac, av, weight):
  mesh = plsc.VectorSubcoreMesh(core_axis_name="c", subcore_axis_name="s")

  def chunk_scratch(ch):
    return [
        pltpu.VMEM((ch,), jnp.int32),
        pltpu.VMEM((ch,), jnp.int32),
        pltpu.VMEM((ch,), jnp.float32),
        pltpu.VMEM((ch,), jnp.int32),
        pltpu.VMEM((ch,), jnp.int32),
        pltpu.VMEM((ch,), jnp.float32),
        pltpu.SemaphoreType.DMA,
        pltpu.SemaphoreType.DMA,
    ]

  densify = pl.kernel(
      _densify_body,
      out_type=jax.ShapeDtypeStruct((NC * D * N,), jnp.float32),
      mesh=mesh,
      scratch_types=[pltpu.VMEM((N * DPT,), jnp.float32)]
      + chunk_scratch(ECH),
      compiler_params=pltpu.CompilerParams(needs_layout_passes=False),
  )
  fb = densify(fr, fc, fv)

  xwt = pl.pallas_call(
      _matmul_kernel,
      out_shape=jax.ShapeDtypeStruct((O, N), jnp.float32),
  )(weight, fb.reshape(NC, D, N))

  spmm = pl.kernel(
      _spmm_body,
      out_type=jax.ShapeDtypeStruct((NW, N * CPT), jnp.float32),
      mesh=mesh,
      scratch_types=[
          pltpu.VMEM((N * CPT,), jnp.float32),
          pltpu.VMEM((N * CPT,), jnp.float32),
      ] + chunk_scratch(CH) + [pltpu.SemaphoreType.DMA],
      compiler_params=pltpu.CompilerParams(needs_layout_passes=False),
  )
  blocks = spmm(ar, ac, av, xwt.reshape(-1))
  return blocks.reshape(NW, CPT, N).transpose(2, 0, 1).reshape(N, O)


def kernel(feat_rows, feat_cols, feat_values, adj_row, adj_col, adj_values,
           weight):
  return _run(feat_rows, feat_cols, feat_values,
              adj_row, adj_col, adj_values, weight)
